# Initial kernel scaffold; baseline (speedup 1.0000x reference)
import jax
import jax.numpy as jnp
from jax import lax
from jax.experimental import pallas as pl
from jax.experimental.pallas import tpu as pltpu
from jax.experimental.pallas import tpu_sc as plsc

N = 10000
D = 128
E = 320000
NC = 2
NS = 16
NW = NC * NS
EPW = E // NW
CHUNK = 128
NFULL = EPW // CHUNK
TAIL = EPW - NFULL * CHUNK
RPW = N // NS
DEG_W = 16

_MESH = plsc.VectorSubcoreMesh(core_axis_name="c", subcore_axis_name="s")


def _spmv_body(src_hbm, sidx_hbm, didx_hbm, zrows_hbm, out_hbm,
               acc, sidx_v, didx_v, rows_v, gsem):
    c = lax.axis_index("c")
    s = lax.axis_index("s")
    wid = s * NC + c
    r0 = s * RPW
    pltpu.sync_copy(zrows_hbm, acc.at[pl.ds(r0, RPW)])
    plsc.subcore_barrier()
    ebase = wid * EPW

    def do_chunk(base, size):
        pltpu.sync_copy(sidx_hbm.at[pl.ds(base, size)], sidx_v.at[pl.ds(0, size)])
        pltpu.sync_copy(didx_hbm.at[pl.ds(base, size)], didx_v.at[pl.ds(0, size)])
        pltpu.async_copy(src_hbm.at[sidx_v.at[pl.ds(0, size)]],
                         rows_v.at[pl.ds(0, size)], gsem).wait()
        pltpu.sync_copy(rows_v.at[pl.ds(0, size)],
                        acc.at[didx_v.at[pl.ds(0, size)]], add=True)

    def step(n, carry):
        do_chunk(ebase + n * CHUNK, CHUNK)
        return carry

    lax.fori_loop(0, NFULL, step, 0)
    do_chunk(ebase + NFULL * CHUNK, TAIL)
    plsc.subcore_barrier()
    pltpu.sync_copy(acc.at[pl.ds(r0, RPW)], out_hbm.at[c, pl.ds(r0, RPW)])


_spmv = pl.kernel(
    _spmv_body,
    out_type=jax.ShapeDtypeStruct((NC, N, D), jnp.float32),
    mesh=_MESH,
    scratch_types=[
        pltpu.VMEM_SHARED((N, D), jnp.float32),
        pltpu.VMEM((CHUNK,), jnp.int32),
        pltpu.VMEM((CHUNK,), jnp.int32),
        pltpu.VMEM((CHUNK, D), jnp.float32),
        pltpu.SemaphoreType.DMA,
    ],
)


def _deg_body(uidx_hbm, iidx_hbm, ones_hbm, zdeg_hbm, out_hbm,
              accu, acci, uidx_v, iidx_v, ones_v):
    c = lax.axis_index("c")
    s = lax.axis_index("s")
    wid = s * NC + c
    r0 = s * RPW
    pltpu.sync_copy(zdeg_hbm, accu.at[pl.ds(r0, RPW)])
    pltpu.sync_copy(zdeg_hbm, acci.at[pl.ds(r0, RPW)])
    pltpu.sync_copy(ones_hbm, ones_v)
    plsc.subcore_barrier()
    ebase = wid * EPW

    def do_chunk(base, size):
        pltpu.sync_copy(uidx_hbm.at[pl.ds(base, size)], uidx_v.at[pl.ds(0, size)])
        pltpu.sync_copy(iidx_hbm.at[pl.ds(base, size)], iidx_v.at[pl.ds(0, size)])
        pltpu.sync_copy(ones_v.at[pl.ds(0, size)],
                        accu.at[uidx_v.at[pl.ds(0, size)]], add=True)
        pltpu.sync_copy(ones_v.at[pl.ds(0, size)],
                        acci.at[iidx_v.at[pl.ds(0, size)]], add=True)

    def step(n, carry):
        do_chunk(ebase + n * CHUNK, CHUNK)
        return carry

    lax.fori_loop(0, NFULL, step, 0)
    do_chunk(ebase + NFULL * CHUNK, TAIL)
    plsc.subcore_barrier()
    pltpu.sync_copy(accu.at[pl.ds(r0, RPW)], out_hbm.at[c, 0, pl.ds(r0, RPW)])
    pltpu.sync_copy(acci.at[pl.ds(r0, RPW)], out_hbm.at[c, 1, pl.ds(r0, RPW)])


_deg = pl.kernel(
    _deg_body,
    out_type=jax.ShapeDtypeStruct((NC, 2, N, DEG_W), jnp.float32),
    mesh=_MESH,
    scratch_types=[
        pltpu.VMEM_SHARED((N, DEG_W), jnp.float32),
        pltpu.VMEM_SHARED((N, DEG_W), jnp.float32),
        pltpu.VMEM((CHUNK,), jnp.int32),
        pltpu.VMEM((CHUNK,), jnp.int32),
        pltpu.VMEM((CHUNK, DEG_W), jnp.float32),
    ],
)


def _combine_body(p_ref, d_ref, o_ref):
    ssum = p_ref[0] + p_ref[1]
    deg = d_ref[0, :, :1] + d_ref[1, :, :1]
    o_ref[...] = ssum / jnp.maximum(deg, 1.0)


_BR = 1000


def _combine(p, dpair):
    return pl.pallas_call(
        _combine_body,
        out_shape=jax.ShapeDtypeStruct((N, D), jnp.float32),
        grid=(N // _BR,),
        in_specs=[
            pl.BlockSpec((NC, _BR, D), lambda j: (0, j, 0)),
            pl.BlockSpec((NC, _BR, DEG_W), lambda j: (0, j, 0)),
        ],
        out_specs=pl.BlockSpec((_BR, D), lambda j: (j, 0)),
    )(p, dpair)


def kernel(user_emb, item_emb, edge_index):
    u = edge_index[0].astype(jnp.int32)
    i = edge_index[1].astype(jnp.int32)
    zrows = jnp.zeros((RPW, D), jnp.float32)
    zdeg = jnp.zeros((RPW, DEG_W), jnp.float32)
    ones = jnp.ones((CHUNK, DEG_W), jnp.float32)
    degs = _deg(u, i, ones, zdeg)
    du = degs[:, 0]
    di = degs[:, 1]
    h_u, h_i = user_emb, item_emb
    for _ in range(2):
        rst = _combine(_spmv(h_u, u, i, zrows), di)
        nu = _combine(_spmv(rst, i, u, zrows), du)
        rs = _combine(_spmv(h_i, i, u, zrows), du)
        ni = _combine(_spmv(rs, u, i, zrows), di)
        h_u, h_i = nu, ni
    return jnp.stack([h_u, h_i], axis=0)


# speedup vs baseline: 5.5473x; 5.5473x over previous
import jax
import jax.numpy as jnp
from jax import lax
from jax.experimental import pallas as pl
from jax.experimental.pallas import tpu as pltpu
from jax.experimental.pallas import tpu_sc as plsc

N = 10000
D = 128
E = 320000
NC = 2
NS = 16
NW = NC * NS
CHUNK = 128
NCH = E // CHUNK
RST = 624
RLEFT = N - NS * RST
DEG_W = 128

_MESH = plsc.VectorSubcoreMesh(core_axis_name="c", subcore_axis_name="s")


def _spmv_body(src_hbm, sidx_hbm, didx_hbm, zrows_hbm, out_hbm,
               acc, sidx_v, didx_v, rows_v, gsem):
    c = lax.axis_index("c")
    s = lax.axis_index("s")
    wid = s * NC + c
    r0 = s * RST
    pltpu.sync_copy(zrows_hbm.at[pl.ds(0, RST)], acc.at[pl.ds(r0, RST)])

    @pl.when(s == 0)
    def _():
        pltpu.sync_copy(zrows_hbm.at[pl.ds(0, RLEFT)],
                        acc.at[pl.ds(NS * RST, RLEFT)])

    plsc.subcore_barrier()
    cs = (wid * NCH) // NW
    ce = ((wid + 1) * NCH) // NW

    def step(n, carry):
        base = n * CHUNK
        pltpu.sync_copy(sidx_hbm.at[pl.ds(base, CHUNK)], sidx_v)
        pltpu.sync_copy(didx_hbm.at[pl.ds(base, CHUNK)], didx_v)
        pltpu.async_copy(src_hbm.at[sidx_v], rows_v, gsem).wait()
        pltpu.sync_copy(rows_v, acc.at[didx_v], add=True)
        return carry

    lax.fori_loop(cs, ce, step, 0)
    plsc.subcore_barrier()
    pltpu.sync_copy(acc.at[pl.ds(r0, RST)], out_hbm.at[c, pl.ds(r0, RST)])

    @pl.when(s == 0)
    def _():
        pltpu.sync_copy(acc.at[pl.ds(NS * RST, RLEFT)],
                        out_hbm.at[c, pl.ds(NS * RST, RLEFT)])


_spmv = pl.kernel(
    _spmv_body,
    out_type=jax.ShapeDtypeStruct((NC, N, D), jnp.float32),
    mesh=_MESH,
    scratch_types=[
        pltpu.VMEM_SHARED((N, D), jnp.float32),
        pltpu.VMEM((CHUNK,), jnp.int32),
        pltpu.VMEM((CHUNK,), jnp.int32),
        pltpu.VMEM((CHUNK, D), jnp.float32),
        pltpu.SemaphoreType.DMA,
    ],
)


def _deg_body(uidx_hbm, iidx_hbm, ones_hbm, zrows_hbm, out_hbm,
              acc, idx_v, ones_v):
    c = lax.axis_index("c")
    s = lax.axis_index("s")
    wid = s * NC + c
    r0 = s * RST
    pltpu.sync_copy(ones_hbm, ones_v)
    cs = (wid * NCH) // NW
    ce = ((wid + 1) * NCH) // NW

    for phase, idx_hbm in enumerate((uidx_hbm, iidx_hbm)):
        pltpu.sync_copy(zrows_hbm.at[pl.ds(0, RST)], acc.at[pl.ds(r0, RST)])

        @pl.when(s == 0)
        def _():
            pltpu.sync_copy(zrows_hbm.at[pl.ds(0, RLEFT)],
                            acc.at[pl.ds(NS * RST, RLEFT)])

        plsc.subcore_barrier()

        def step(n, carry):
            pltpu.sync_copy(idx_hbm.at[pl.ds(n * CHUNK, CHUNK)], idx_v)
            pltpu.sync_copy(ones_v, acc.at[idx_v], add=True)
            return carry

        lax.fori_loop(cs, ce, step, 0)
        plsc.subcore_barrier()
        pltpu.sync_copy(acc.at[pl.ds(r0, RST)],
                        out_hbm.at[c, phase, pl.ds(r0, RST)])

        @pl.when(s == 0)
        def _():
            pltpu.sync_copy(acc.at[pl.ds(NS * RST, RLEFT)],
                            out_hbm.at[c, phase, pl.ds(NS * RST, RLEFT)])


_deg = pl.kernel(
    _deg_body,
    out_type=jax.ShapeDtypeStruct((NC, 2, N, DEG_W), jnp.float32),
    mesh=_MESH,
    scratch_types=[
        pltpu.VMEM_SHARED((N, DEG_W), jnp.float32),
        pltpu.VMEM((CHUNK,), jnp.int32),
        pltpu.VMEM((CHUNK, DEG_W), jnp.float32),
    ],
)


def _combine_body(p_ref, d_ref, o_ref):
    ssum = p_ref[0] + p_ref[1]
    deg = d_ref[0, :, :1] + d_ref[1, :, :1]
    o_ref[...] = ssum / jnp.maximum(deg, 1.0)


_BR = 1000


def _combine(p, dpair):
    return pl.pallas_call(
        _combine_body,
        out_shape=jax.ShapeDtypeStruct((N, D), jnp.float32),
        grid=(N // _BR,),
        in_specs=[
            pl.BlockSpec((NC, _BR, D), lambda j: (0, j, 0)),
            pl.BlockSpec((NC, _BR, DEG_W), lambda j: (0, j, 0)),
        ],
        out_specs=pl.BlockSpec((_BR, D), lambda j: (j, 0)),
    )(p, dpair)


def kernel(user_emb, item_emb, edge_index):
    u = edge_index[0].astype(jnp.int32)
    i = edge_index[1].astype(jnp.int32)
    zrows = jnp.zeros((RST, D), jnp.float32)
    ones = jnp.ones((CHUNK, DEG_W), jnp.float32)
    degs = _deg(u, i, ones, zrows)
    du = degs[:, 0]
    di = degs[:, 1]
    h_u, h_i = user_emb, item_emb
    for _ in range(2):
        rst = _combine(_spmv(h_u, u, i, zrows), di)
        nu = _combine(_spmv(rst, i, u, zrows), du)
        rs = _combine(_spmv(h_i, i, u, zrows), du)
        ni = _combine(_spmv(rs, u, i, zrows), di)
        h_u, h_i = nu, ni
    return jnp.stack([h_u, h_i], axis=0)
